# baseline (speedup 1.0000x reference)
import jax
import jax.numpy as jnp
from jax.experimental import pallas as pl
from jax.experimental.pallas import tpu as pltpu

_TT = 512


def _top2_combine(logits):
    E = logits.shape[-1]
    eids = jax.lax.broadcasted_iota(jnp.int32, logits.shape, 1)
    m1 = jnp.max(logits, axis=1, keepdims=True)
    i1 = jnp.min(jnp.where(logits == m1, eids, E), axis=1, keepdims=True)
    mask1 = eids == i1
    masked = jnp.where(mask1, -jnp.inf, logits)
    m2 = jnp.max(masked, axis=1, keepdims=True)
    i2 = jnp.min(jnp.where(masked == m2, eids, E), axis=1, keepdims=True)
    mask2 = eids == i2
    e2 = jnp.exp(m2 - m1)
    w1 = 1.0 / (1.0 + e2)
    w2 = e2 / (1.0 + e2)
    return w1 * mask1.astype(logits.dtype) + w2 * mask2.astype(logits.dtype)


def _moe_kernel(x_ref, wg_ref, we_ref, be_ref, out_ref):
    x = x_ref[...]
    logits = jnp.dot(x, wg_ref[...], preferred_element_type=jnp.float32)
    combine = _top2_combine(logits)
    E = logits.shape[-1]
    xb = x.astype(jnp.bfloat16)
    xcat = jnp.concatenate(
        [xb * combine[:, e:e + 1].astype(jnp.bfloat16) for e in range(E)],
        axis=1)
    acc = jnp.dot(combine, be_ref[...], preferred_element_type=jnp.float32)
    acc = acc + jnp.dot(xcat, we_ref[...], preferred_element_type=jnp.float32)
    out_ref[...] = acc


def _cast_kernel(we_ref, out_ref):
    out_ref[...] = we_ref[...].astype(jnp.bfloat16)


def kernel(inputs, Wg, We, be):
    D = inputs.shape[-1]
    E = We.shape[0]
    xf = inputs.reshape(-1, D)
    T = xf.shape[0]
    we_flat = We.astype(jnp.bfloat16).reshape(E * D, D)
    out = pl.pallas_call(
        _moe_kernel,
        grid=(T // _TT,),
        in_specs=[
            pl.BlockSpec((_TT, D), lambda i: (i, 0)),
            pl.BlockSpec(Wg.shape, lambda i: (0, 0)),
            pl.BlockSpec((E * D, D), lambda i: (0, 0)),
            pl.BlockSpec(be.shape, lambda i: (0, 0)),
        ],
        out_specs=pl.BlockSpec((_TT, D), lambda i: (i, 0)),
        out_shape=jax.ShapeDtypeStruct((T, D), inputs.dtype),
    )(xf, Wg, we_flat, be)
    return out.reshape(inputs.shape)


# speedup vs baseline: 1.0453x; 1.0453x over previous
import jax
import jax.numpy as jnp
from jax.experimental import pallas as pl
from jax.experimental.pallas import tpu as pltpu

_TT = 256


def _top2_combine(logits):
    E = logits.shape[-1]
    eids = jax.lax.broadcasted_iota(jnp.int32, logits.shape, 1)
    m1 = jnp.max(logits, axis=1, keepdims=True)
    i1 = jnp.min(jnp.where(logits == m1, eids, E), axis=1, keepdims=True)
    mask1 = eids == i1
    masked = jnp.where(mask1, -jnp.inf, logits)
    m2 = jnp.max(masked, axis=1, keepdims=True)
    i2 = jnp.min(jnp.where(masked == m2, eids, E), axis=1, keepdims=True)
    mask2 = eids == i2
    e2 = jnp.exp(m2 - m1)
    w1 = 1.0 / (1.0 + e2)
    w2 = e2 / (1.0 + e2)
    return w1 * mask1.astype(logits.dtype) + w2 * mask2.astype(logits.dtype)


def _moe_kernel(x_ref, wg_ref, we_ref, be_ref, out_ref, webf_ref):
    @pl.when(pl.program_id(0) == 0)
    def _cast():
        webf_ref[...] = we_ref[...].astype(jnp.bfloat16)

    x = x_ref[...]
    logits = jnp.dot(x, wg_ref[...], preferred_element_type=jnp.float32)
    combine = _top2_combine(logits)
    E = logits.shape[-1]
    xb = x.astype(jnp.bfloat16)
    xcat = jnp.concatenate(
        [xb * combine[:, e:e + 1].astype(jnp.bfloat16) for e in range(E)],
        axis=1)
    acc = jnp.dot(combine, be_ref[...], preferred_element_type=jnp.float32)
    acc = acc + jnp.dot(xcat, webf_ref[...], preferred_element_type=jnp.float32)
    out_ref[...] = acc


def kernel(inputs, Wg, We, be):
    D = inputs.shape[-1]
    E = We.shape[0]
    xf = inputs.reshape(-1, D)
    T = xf.shape[0]
    we_flat = We.reshape(E * D, D)
    out = pl.pallas_call(
        _moe_kernel,
        grid=(T // _TT,),
        in_specs=[
            pl.BlockSpec((_TT, D), lambda i: (i, 0)),
            pl.BlockSpec(Wg.shape, lambda i: (0, 0)),
            pl.BlockSpec((E * D, D), lambda i: (0, 0)),
            pl.BlockSpec(be.shape, lambda i: (0, 0)),
        ],
        out_specs=pl.BlockSpec((_TT, D), lambda i: (i, 0)),
        out_shape=jax.ShapeDtypeStruct((T, D), inputs.dtype),
        scratch_shapes=[pltpu.VMEM((E * D, D), jnp.bfloat16)],
        compiler_params=pltpu.CompilerParams(
            vmem_limit_bytes=120 * 1024 * 1024),
    )(xf, Wg, we_flat, be)
    return out.reshape(inputs.shape)


# speedup vs baseline: 1.1068x; 1.0588x over previous
import jax
import jax.numpy as jnp
from jax.experimental import pallas as pl
from jax.experimental.pallas import tpu as pltpu

_TT = 512


def _top2_combine(logits):
    E = logits.shape[-1]
    eids = jax.lax.broadcasted_iota(jnp.int32, logits.shape, 1)
    m1 = jnp.max(logits, axis=1, keepdims=True)
    i1 = jnp.min(jnp.where(logits == m1, eids, E), axis=1, keepdims=True)
    mask1 = eids == i1
    masked = jnp.where(mask1, -jnp.inf, logits)
    m2 = jnp.max(masked, axis=1, keepdims=True)
    i2 = jnp.min(jnp.where(masked == m2, eids, E), axis=1, keepdims=True)
    mask2 = eids == i2
    e2 = jnp.exp(m2 - m1)
    w1 = 1.0 / (1.0 + e2)
    w2 = e2 / (1.0 + e2)
    return w1 * mask1.astype(logits.dtype) + w2 * mask2.astype(logits.dtype)


def _moe_kernel(x_ref, wg_ref, we_ref, be_ref, out_ref, webf_ref):
    @pl.when(pl.program_id(0) == 0)
    def _cast():
        webf_ref[...] = we_ref[...].astype(jnp.bfloat16)

    x = x_ref[...]
    logits = jnp.dot(x, wg_ref[...], preferred_element_type=jnp.float32)
    combine = _top2_combine(logits)
    E = logits.shape[-1]
    D = x.shape[-1]
    xb = x.astype(jnp.bfloat16)
    acc = jnp.dot(combine, be_ref[...], preferred_element_type=jnp.float32)
    for e in range(E):
        xe = xb * combine[:, e:e + 1].astype(jnp.bfloat16)
        acc = acc + jnp.dot(xe, webf_ref[e * D:(e + 1) * D, :],
                            preferred_element_type=jnp.float32)
    out_ref[...] = acc


def kernel(inputs, Wg, We, be):
    D = inputs.shape[-1]
    E = We.shape[0]
    xf = inputs.reshape(-1, D)
    T = xf.shape[0]
    we_flat = We.reshape(E * D, D)
    out = pl.pallas_call(
        _moe_kernel,
        grid=(T // _TT,),
        in_specs=[
            pl.BlockSpec((_TT, D), lambda i: (i, 0)),
            pl.BlockSpec(Wg.shape, lambda i: (0, 0)),
            pl.BlockSpec((E * D, D), lambda i: (0, 0)),
            pl.BlockSpec(be.shape, lambda i: (0, 0)),
        ],
        out_specs=pl.BlockSpec((_TT, D), lambda i: (i, 0)),
        out_shape=jax.ShapeDtypeStruct((T, D), inputs.dtype),
        scratch_shapes=[pltpu.VMEM((E * D, D), jnp.bfloat16)],
        compiler_params=pltpu.CompilerParams(
            vmem_limit_bytes=120 * 1024 * 1024),
    )(xf, Wg, we_flat, be)
    return out.reshape(inputs.shape)


# speedup vs baseline: 1.1245x; 1.0160x over previous
import jax
import jax.numpy as jnp
from jax.experimental import pallas as pl
from jax.experimental.pallas import tpu as pltpu

_TT = 512


def _top2_combine(logits):
    E = logits.shape[-1]
    eids = jax.lax.broadcasted_iota(jnp.int32, logits.shape, 1)
    m1 = jnp.max(logits, axis=1, keepdims=True)
    i1 = jnp.min(jnp.where(logits == m1, eids, E), axis=1, keepdims=True)
    mask1 = eids == i1
    masked = jnp.where(mask1, -jnp.inf, logits)
    m2 = jnp.max(masked, axis=1, keepdims=True)
    i2 = jnp.min(jnp.where(masked == m2, eids, E), axis=1, keepdims=True)
    mask2 = eids == i2
    e2 = jnp.exp(m2 - m1)
    w1 = 1.0 / (1.0 + e2)
    w2 = e2 / (1.0 + e2)
    return w1 * mask1.astype(logits.dtype) + w2 * mask2.astype(logits.dtype)


def _moe_kernel(x_ref, wg_ref, we_ref, be_ref, out_ref):
    x = x_ref[...]
    logits = jnp.dot(x, wg_ref[...], preferred_element_type=jnp.float32)
    combine = _top2_combine(logits)
    E = logits.shape[-1]
    D = x.shape[-1]
    xb = x.astype(jnp.bfloat16)
    acc = jnp.dot(combine, be_ref[...], preferred_element_type=jnp.float32)
    for e in range(E):
        xe = xb * combine[:, e:e + 1].astype(jnp.bfloat16)
        acc = acc + jnp.dot(xe, we_ref[e * D:(e + 1) * D, :].astype(jnp.bfloat16),
                            preferred_element_type=jnp.float32)
    out_ref[...] = acc


def kernel(inputs, Wg, We, be):
    D = inputs.shape[-1]
    E = We.shape[0]
    xf = inputs.reshape(-1, D)
    T = xf.shape[0]
    we_flat = We.reshape(E * D, D)
    out = pl.pallas_call(
        _moe_kernel,
        grid=(T // _TT,),
        in_specs=[
            pl.BlockSpec((_TT, D), lambda i: (i, 0)),
            pl.BlockSpec(Wg.shape, lambda i: (0, 0)),
            pl.BlockSpec((E * D, D), lambda i: (0, 0)),
            pl.BlockSpec(be.shape, lambda i: (0, 0)),
        ],
        out_specs=pl.BlockSpec((_TT, D), lambda i: (i, 0)),
        out_shape=jax.ShapeDtypeStruct((T, D), inputs.dtype),
        compiler_params=pltpu.CompilerParams(
            vmem_limit_bytes=120 * 1024 * 1024),
    )(xf, Wg, we_flat, be)
    return out.reshape(inputs.shape)


# speedup vs baseline: 1.2413x; 1.1039x over previous
import jax
import jax.numpy as jnp
from jax.experimental import pallas as pl
from jax.experimental.pallas import tpu as pltpu

_TT = 512


def _top2_combine(logits):
    E = logits.shape[-1]
    eids = jax.lax.broadcasted_iota(jnp.int32, logits.shape, 1)
    m1 = jnp.max(logits, axis=1, keepdims=True)
    i1 = jnp.min(jnp.where(logits == m1, eids, E), axis=1, keepdims=True)
    mask1 = eids == i1
    masked = jnp.where(mask1, -jnp.inf, logits)
    m2 = jnp.max(masked, axis=1, keepdims=True)
    i2 = jnp.min(jnp.where(masked == m2, eids, E), axis=1, keepdims=True)
    mask2 = eids == i2
    e2 = jnp.exp(m2 - m1)
    w1 = 1.0 / (1.0 + e2)
    w2 = e2 / (1.0 + e2)
    return w1 * mask1.astype(logits.dtype) + w2 * mask2.astype(logits.dtype)


def _moe_kernel(x_ref, wg_ref, we_ref, be_ref, out_ref):
    x = x_ref[...]
    logits = jnp.dot(x, wg_ref[...], preferred_element_type=jnp.float32)
    combine = _top2_combine(logits)
    E = logits.shape[-1]
    D = x.shape[-1]
    xb = x.astype(jnp.bfloat16)
    acc = jnp.dot(combine, be_ref[...], preferred_element_type=jnp.float32)
    for e in range(E):
        ye = jnp.dot(xb, we_ref[e].astype(jnp.bfloat16),
                     preferred_element_type=jnp.float32)
        acc = acc + combine[:, e:e + 1] * ye
    out_ref[...] = acc


def kernel(inputs, Wg, We, be):
    D = inputs.shape[-1]
    E = We.shape[0]
    xf = inputs.reshape(-1, D)
    T = xf.shape[0]

    out = pl.pallas_call(
        _moe_kernel,
        grid=(T // _TT,),
        in_specs=[
            pl.BlockSpec((_TT, D), lambda i: (i, 0)),
            pl.BlockSpec(Wg.shape, lambda i: (0, 0)),
            pl.BlockSpec(We.shape, lambda i: (0, 0, 0)),
            pl.BlockSpec(be.shape, lambda i: (0, 0)),
        ],
        out_specs=pl.BlockSpec((_TT, D), lambda i: (i, 0)),
        out_shape=jax.ShapeDtypeStruct((T, D), inputs.dtype),
    )(xf, Wg, We, be)
    return out.reshape(inputs.shape)
